# Initial kernel scaffold; baseline (speedup 1.0000x reference)
import jax
import jax.numpy as jnp
from jax.experimental import pallas as pl


def kernel(input_, idx_list, W, b):
    raise NotImplementedError("write your pallas kernel here")


# speedup vs baseline: 5.7853x; 5.7853x over previous
import functools

import jax
import jax.numpy as jnp
from jax.experimental import pallas as pl
from jax.experimental.pallas import tpu as pltpu

BT = 512


def _moe_block(owner_ref, x_ref, wt_ref, b_ref, out_ref):
    s = pl.program_id(0)
    e = owner_ref[s]
    x = x_ref[0]
    wt = wt_ref[0]
    y = jnp.dot(x, wt, preferred_element_type=jnp.float32) + b_ref[0]
    tiled = jnp.concatenate([y] * 8, axis=-1)
    lane_grp = jax.lax.broadcasted_iota(jnp.int32, tiled.shape, 1) // 64
    out_ref[0] = jnp.where(lane_grp == e, tiled, 0.0)


@jax.jit
def kernel(input_, idx_list, W, b):
    bs, seq_len, d_model = input_.shape
    world_size, out_per, _ = W.shape
    d_out = world_size * out_per

    owner = jnp.zeros((bs,), jnp.int32).at[idx_list.reshape(-1)].set(
        jnp.repeat(jnp.arange(world_size, dtype=jnp.int32), idx_list.shape[1]))

    wt = jnp.transpose(W, (0, 2, 1))
    b_seq = b[owner][:, None, :]

    grid = (bs, seq_len // BT)
    out = pl.pallas_call(
        _moe_block,
        grid_spec=pltpu.PrefetchScalarGridSpec(
            num_scalar_prefetch=1,
            grid=grid,
            in_specs=[
                pl.BlockSpec((1, BT, d_model), lambda s, t, own: (s, t, 0)),
                pl.BlockSpec((1, d_model, out_per), lambda s, t, own: (own[s], 0, 0)),
                pl.BlockSpec((1, 1, out_per), lambda s, t, own: (s, 0, 0)),
            ],
            out_specs=pl.BlockSpec((1, BT, d_out), lambda s, t, own: (s, t, 0)),
        ),
        out_shape=jax.ShapeDtypeStruct((bs, seq_len, d_out), input_.dtype),
    )(owner, input_, wt, b_seq)
    return out


# speedup vs baseline: 7.4012x; 1.2793x over previous
import functools

import jax
import jax.numpy as jnp
from jax.experimental import pallas as pl
from jax.experimental.pallas import tpu as pltpu

BT = 1024


def _moe_block(owner_ref, x_ref, wt_ref, b_ref, out_ref):
    s = pl.program_id(0)
    e = owner_ref[s]
    x = x_ref[0]
    wt = wt_ref[0]
    y = jnp.dot(x, wt, preferred_element_type=jnp.float32) + b_ref[0]
    tiled = jnp.concatenate([y] * 8, axis=-1)
    lane_grp = jax.lax.broadcasted_iota(jnp.int32, tiled.shape, 1) // 64
    out_ref[0] = jnp.where(lane_grp == e, tiled, 0.0)


@jax.jit
def kernel(input_, idx_list, W, b):
    bs, seq_len, d_model = input_.shape
    world_size, out_per, _ = W.shape
    d_out = world_size * out_per

    owner = jnp.zeros((bs,), jnp.int32).at[idx_list.reshape(-1)].set(
        jnp.repeat(jnp.arange(world_size, dtype=jnp.int32), idx_list.shape[1]))

    wt = jnp.transpose(W, (0, 2, 1))
    b_seq = b[owner][:, None, :]

    grid = (bs, seq_len // BT)
    out = pl.pallas_call(
        _moe_block,
        grid_spec=pltpu.PrefetchScalarGridSpec(
            num_scalar_prefetch=1,
            grid=grid,
            in_specs=[
                pl.BlockSpec((1, BT, d_model), lambda s, t, own: (s, t, 0)),
                pl.BlockSpec((1, d_model, out_per), lambda s, t, own: (own[s], 0, 0)),
                pl.BlockSpec((1, 1, out_per), lambda s, t, own: (s, 0, 0)),
            ],
            out_specs=pl.BlockSpec((1, BT, d_out), lambda s, t, own: (s, t, 0)),
        ),
        out_shape=jax.ShapeDtypeStruct((bs, seq_len, d_out), input_.dtype),
    )(owner, input_, wt, b_seq)
    return out


# speedup vs baseline: 7.7977x; 1.0536x over previous
import functools

import jax
import jax.numpy as jnp
from jax.experimental import pallas as pl
from jax.experimental.pallas import tpu as pltpu

BT = 2048


def _moe_block(owner_ref, x_ref, wt_ref, b_ref, out_ref):
    s = pl.program_id(0)
    e = owner_ref[s]
    x = x_ref[0]
    wt = wt_ref[0]
    y = jnp.dot(x, wt, preferred_element_type=jnp.float32) + b_ref[0]
    tiled = jnp.concatenate([y] * 8, axis=-1)
    lane_grp = jax.lax.broadcasted_iota(jnp.int32, tiled.shape, 1) // 64
    out_ref[0] = jnp.where(lane_grp == e, tiled, 0.0)


@jax.jit
def kernel(input_, idx_list, W, b):
    bs, seq_len, d_model = input_.shape
    world_size, out_per, _ = W.shape
    d_out = world_size * out_per

    owner = jnp.zeros((bs,), jnp.int32).at[idx_list.reshape(-1)].set(
        jnp.repeat(jnp.arange(world_size, dtype=jnp.int32), idx_list.shape[1]))

    wt = jnp.transpose(W, (0, 2, 1))
    b_seq = b[owner][:, None, :]

    grid = (bs, seq_len // BT)
    out = pl.pallas_call(
        _moe_block,
        grid_spec=pltpu.PrefetchScalarGridSpec(
            num_scalar_prefetch=1,
            grid=grid,
            in_specs=[
                pl.BlockSpec((1, BT, d_model), lambda s, t, own: (s, t, 0)),
                pl.BlockSpec((1, d_model, out_per), lambda s, t, own: (own[s], 0, 0)),
                pl.BlockSpec((1, 1, out_per), lambda s, t, own: (s, 0, 0)),
            ],
            out_specs=pl.BlockSpec((1, BT, d_out), lambda s, t, own: (s, t, 0)),
        ),
        out_shape=jax.ShapeDtypeStruct((bs, seq_len, d_out), input_.dtype),
    )(owner, input_, wt, b_seq)
    return out
